# baseline (speedup 1.0000x reference)
import jax
import jax.numpy as jnp
from jax.experimental import pallas as pl
from jax.experimental.pallas import tpu as pltpu

_BN = 1024


def _cnorm_block(c_ref, cn_ref):
    c = c_ref[...]
    cn_ref[...] = jnp.sum(c * c, axis=0, keepdims=True)


def _kmeans_block(x_ref, c_ref, cn_ref, dist_ref, idx_ref):
    x = x_ref[...]
    k = c_ref.shape[1]
    xn = jnp.sum(x * x, axis=1, keepdims=True)
    xc = jax.lax.dot_general(
        x, c_ref[...], (((1,), (0,)), ((), ())),
        preferred_element_type=jnp.float32,
        precision=jax.lax.Precision.DEFAULT,
    )
    dist = xn - 2.0 * xc + cn_ref[...]
    dist_ref[...] = dist
    lanes = 128
    ngroups = k // lanes
    cols = [dist[:, j * lanes:(j + 1) * lanes] for j in range(ngroups)]
    m = cols[0]
    for cj in cols[1:]:
        m = jnp.minimum(m, cj)
    g = jnp.full(m.shape, ngroups, dtype=jnp.int32)
    for j in reversed(range(ngroups)):
        g = jnp.where(cols[j] == m, jnp.int32(j), g)
    lane_iota = jax.lax.broadcasted_iota(jnp.int32, m.shape, 1)
    kidx = g * lanes + lane_iota
    mrow = jnp.min(m, axis=1, keepdims=True)
    idx_ref[...] = jnp.min(jnp.where(m == mrow, kidx, k), axis=1)


def kernel(x, C):
    n, d = x.shape
    d2, k = C.shape
    assert d == d2
    cn = pl.pallas_call(
        _cnorm_block,
        grid=(1,),
        in_specs=[pl.BlockSpec((d, k), lambda i: (0, 0))],
        out_specs=pl.BlockSpec((1, k), lambda i: (0, 0)),
        out_shape=jax.ShapeDtypeStruct((1, k), jnp.float32),
    )(C)
    grid = (n // _BN,)
    dist, idx = pl.pallas_call(
        _kmeans_block,
        grid=grid,
        in_specs=[
            pl.BlockSpec((_BN, d), lambda i: (i, 0)),
            pl.BlockSpec((d, k), lambda i: (0, 0)),
            pl.BlockSpec((1, k), lambda i: (0, 0)),
        ],
        out_specs=[
            pl.BlockSpec((_BN, k), lambda i: (i, 0)),
            pl.BlockSpec((_BN,), lambda i: (i,)),
        ],
        out_shape=[
            jax.ShapeDtypeStruct((n, k), jnp.float32),
            jax.ShapeDtypeStruct((n,), jnp.int32),
        ],
        compiler_params=pltpu.CompilerParams(
            dimension_semantics=("parallel",),
        ),
    )(x, C, cn)
    return (idx, dist)


# speedup vs baseline: 1.0673x; 1.0673x over previous
import jax
import jax.numpy as jnp
from jax.experimental import pallas as pl
from jax.experimental.pallas import tpu as pltpu

_BN = 1024


def _kmeans_block(x_ref, c_ref, dist_ref, idx_ref, cn_ref):
    @pl.when(pl.program_id(0) == 0)
    def _():
        c0 = c_ref[...]
        cn_ref[...] = jnp.sum(c0 * c0, axis=0, keepdims=True)
    x = x_ref[...]
    k = c_ref.shape[1]
    xn = jnp.sum(x * x, axis=1, keepdims=True)
    xc = jax.lax.dot_general(
        x, c_ref[...], (((1,), (0,)), ((), ())),
        preferred_element_type=jnp.float32,
        precision=jax.lax.Precision.DEFAULT,
    )
    dist = xn - 2.0 * xc + cn_ref[...]
    dist_ref[...] = dist
    lanes = 128
    ngroups = k // lanes
    cols = [dist[:, j * lanes:(j + 1) * lanes] for j in range(ngroups)]
    m = cols[0]
    for cj in cols[1:]:
        m = jnp.minimum(m, cj)
    g = jnp.full(m.shape, ngroups, dtype=jnp.int32)
    for j in reversed(range(ngroups)):
        g = jnp.where(cols[j] == m, jnp.int32(j), g)
    lane_iota = jax.lax.broadcasted_iota(jnp.int32, m.shape, 1)
    kidx = g * lanes + lane_iota
    mrow = jnp.min(m, axis=1, keepdims=True)
    idx_ref[...] = jnp.min(jnp.where(m == mrow, kidx, k), axis=1)


def kernel(x, C):
    n, d = x.shape
    d2, k = C.shape
    assert d == d2
    grid = (n // _BN,)
    dist, idx = pl.pallas_call(
        _kmeans_block,
        grid=grid,
        in_specs=[
            pl.BlockSpec((_BN, d), lambda i: (i, 0)),
            pl.BlockSpec((d, k), lambda i: (0, 0)),
        ],
        out_specs=[
            pl.BlockSpec((_BN, k), lambda i: (i, 0)),
            pl.BlockSpec((_BN,), lambda i: (i,)),
        ],
        out_shape=[
            jax.ShapeDtypeStruct((n, k), jnp.float32),
            jax.ShapeDtypeStruct((n,), jnp.int32),
        ],
        scratch_shapes=[pltpu.VMEM((1, k), jnp.float32)],
    )(x, C)
    return (idx, dist)
